# baseline (speedup 1.0000x reference)
import functools

import jax
import jax.numpy as jnp
from jax import lax
from jax.experimental import pallas as pl
from jax.experimental.pallas import tpu as pltpu
from jax.experimental.pallas import tpu_sc as plsc

B, N, H, W, R, P, C = 8, 8192, 512, 512, 2, 5, 5
NC, NS, L = 2, 16, 16
NT = NC * NS
ROWS = H // NT
GR = ROWS + 2 * R
WP = W + 2 * R
GSZ = GR * WP * C
PATCH = C * P * P
OUTW = 128
NGRP = N // L

_mesh = plsc.VectorSubcoreMesh(
    core_axis_name="c", subcore_axis_name="s", num_cores=NC, num_subcores=NS)


@functools.partial(
    pl.kernel,
    out_type=jax.ShapeDtypeStruct((B * N, OUTW), jnp.float32),
    mesh=_mesh,
    compiler_params=pltpu.CompilerParams(
        needs_layout_passes=False, use_tc_tiling_on_sc=False),
    scratch_types=[
        pltpu.VMEM((GSZ,), jnp.float32),
        pltpu.VMEM((2 * N,), jnp.int32),
        pltpu.VMEM((2 * N,), jnp.float32),
        pltpu.VMEM((N,), jnp.float32),
        pltpu.VMEM((N,), jnp.float32),
        pltpu.VMEM((N,), jnp.int32),
        pltpu.VMEM((GR * W,), jnp.float32),
        pltpu.VMEM((2 * L, OUTW), jnp.float32),
        pltpu.VMEM((L, OUTW + 1), jnp.float32),
        pltpu.VMEM((L,), jnp.int32),
        pltpu.VMEM((L,), jnp.int32),
        pltpu.SMEM((1,), jnp.int32),
        pltpu.SMEM((1,), jnp.int32),
        pltpu.SemaphoreType.DMA,
        pltpu.SemaphoreType.DMA,
        pltpu.SemaphoreType.DMA,
    ],
)
def _sc_world_model(xy_hbm, sp_hbm, e_hbm, al_hbm, feed_hbm,
                    out_hbm, grid, xyv, spv, ev, alv, listv, feedv,
                    stag, stbf, idx0, idx1, cnt_sm, gg_sm, sem_in, sem0,
                    sem1):
    wid = lax.axis_index("s") * NC + lax.axis_index("c")
    lo = wid * ROWS
    iota = lax.iota(jnp.int32, L)
    iota2 = iota * 2
    ones = jnp.ones((L,), jnp.float32)
    zvec = jnp.zeros((L,), jnp.float32)
    gg_sm[0] = 0

    def out_wait(p_is_0_first):
        gg = gg_sm[0]
        p = gg & 1

        @pl.when(jnp.logical_and(gg >= 2, p == 0))
        def _():
            pltpu.make_async_copy(
                stag.at[pl.ds(0, L)], out_hbm.at[idx0], sem0).wait()

        @pl.when(jnp.logical_and(gg >= 2, p == 1))
        def _():
            pltpu.make_async_copy(
                stag.at[pl.ds(L, L)], out_hbm.at[idx1], sem1).wait()

    def batch_body(b, _carry):
        fires = [
            pltpu.async_copy(xy_hbm.at[pl.ds(b * 2 * N, 2 * N)], xyv, sem_in),
            pltpu.async_copy(sp_hbm.at[pl.ds(b * 2 * N, 2 * N)], spv, sem_in),
            pltpu.async_copy(e_hbm.at[pl.ds(b * N, N)], ev, sem_in),
            pltpu.async_copy(al_hbm.at[pl.ds(b * N, N)], alv, sem_in),
        ]
        feed_valid = []
        for l in range(GR):
            gr = lo - R + l
            valid = jnp.logical_and(gr >= 0, gr < H)
            feed_valid.append(valid)

            @pl.when(valid)
            def _(l=l, gr=gr):
                pltpu.async_copy(feed_hbm.at[b, gr],
                                 feedv.at[pl.ds(l * W, W)], sem_in)

        @pl.when(b == 0)
        def _():
            UZ = 25

            def zero_body(i, c):
                base = i * (UZ * L)
                for u in range(UZ):
                    grid[pl.ds(base + u * L, L)] = zvec
                return c
            lax.fori_loop(0, GSZ // (UZ * L), zero_body, 0)

        @pl.when(b > 0)
        def _():
            HZ = 5168
            UZ2 = 17

            def zero_body(i, c):
                base = i * (UZ2 * L)
                for u in range(UZ2):
                    grid[pl.ds(base + u * L, L)] = zvec
                    grid[pl.ds(GSZ - HZ + base + u * L, L)] = zvec
                return c
            lax.fori_loop(0, HZ // (UZ2 * L), zero_body, 0)

        for f in fires:
            f.wait()
        for l in range(GR):
            @pl.when(feed_valid[l])
            def _(l=l):
                pltpu.make_async_copy(feed_hbm.at[b, 0],
                                      feedv.at[pl.ds(l * W, W)], sem_in).wait()

        FCH = 8
        for l in range(GR):
            @pl.when(feed_valid[l])
            def _(l=l):
                for kk0 in range(0, W // L, FCH):
                    vs = [feedv[pl.ds(l * W + (kk0 + u) * L, L)]
                          for u in range(FCH)]
                    for u in range(FCH):
                        gidx = ((l * WP + R + (kk0 + u) * L) * C + 3
                                + iota * C)
                        plsc.store_scatter(grid, [gidx], vs[u])

        cnt_sm[0] = 0

        US = 4

        def scan_body(gq, _c):
            for u in range(US):
                off = (gq * US + u) * L
                idx2 = 2 * off + iota2
                xg = plsc.load_gather(xyv, [idx2])
                pre = jnp.logical_and(xg >= lo - R, xg < lo + ROWS + R)

                @pl.when(jnp.any(pre))
                def _(off=off, idx2=idx2, xg=xg, pre=pre):
                    yg = plsc.load_gather(xyv, [idx2 + 1])
                    ag = alv[pl.ds(off, L)]
                    m = jnp.logical_and(pre, ag > 0.0)
                    base = ((xg - (lo - R)) * WP + yg + R) * C
                    plsc.store_scatter(grid, [base],
                                       plsc.load_gather(spv, [idx2]), mask=m)
                    plsc.store_scatter(grid, [base + 1],
                                       plsc.load_gather(spv, [idx2 + 1]),
                                       mask=m)
                    plsc.store_scatter(grid, [base + 2], ev[pl.ds(off, L)],
                                       mask=m)
                    plsc.store_scatter(grid, [base + 4], ones, mask=m)
                    own = jnp.logical_and(xg >= lo, xg < lo + ROWS)
                    cnt = cnt_sm[0]
                    plsc.store_compressed(listv.at[pl.ds(cnt, L)],
                                          off + iota, mask=own)
                    pc = plsc.all_reduce_population_count(own)
                    cnt_sm[0] = cnt + pc[0]
            return _c

        lax.fori_loop(0, NGRP // US, scan_body, 0)

        kcnt = cnt_sm[0]
        ngrp = (kcnt + (L - 1)) >> 4

        def group_body(g, _c):
            out_wait(None)
            rem = kcnt - g * L
            safe_lane = jnp.minimum(iota, rem - 1)
            ids = plsc.load_gather(listv, [g * L + safe_lane])
            xg = plsc.load_gather(xyv, [ids * 2])
            yg = plsc.load_gather(xyv, [ids * 2 + 1])
            ag = plsc.load_gather(alv, [ids])
            p = gg_sm[0] & 1
            base = ((xg - lo) * WP + yg) * C
            GCH = 8
            for k0 in range(0, PATCH, GCH):
                ks = range(k0, min(k0 + GCH, PATCH))
                vs = []
                for k in ks:
                    c, ij = k // (P * P), k % (P * P)
                    i, j = ij // P, ij % P
                    d = (i * WP + j) * C + c
                    vs.append(plsc.load_gather(grid, [base + d]))
                for k, v in zip(ks, vs):
                    plsc.store_scatter(
                        stbf, [iota, jnp.full((L,), k, jnp.int32)], v * ag)
            for l0 in range(L):
                for u0 in range(OUTW // L):
                    stag[p * L + l0, pl.ds(u0 * L, L)] = (
                        stbf[l0, pl.ds(u0 * L, L)])
            rowids = b * N + ids

            @pl.when(p == 0)
            def _():
                idx0[...] = rowids
                pltpu.async_copy(stag.at[pl.ds(0, L)], out_hbm.at[idx0], sem0)

            @pl.when(p == 1)
            def _():
                idx1[...] = rowids
                pltpu.async_copy(stag.at[pl.ds(L, L)], out_hbm.at[idx1], sem1)

            gg_sm[0] = gg_sm[0] + 1
            return _c

        lax.fori_loop(0, ngrp, group_body, 0)

        @pl.when(b < B - 1)
        def _():
            def clean_body(g, _c):
                safe_lane = jnp.minimum(iota, kcnt - g * L - 1)
                ids = plsc.load_gather(listv, [g * L + safe_lane])
                xg = plsc.load_gather(xyv, [ids * 2])
                yg = plsc.load_gather(xyv, [ids * 2 + 1])
                base = ((xg - (lo - R)) * WP + yg + R) * C
                plsc.store_scatter(grid, [base], zvec)
                plsc.store_scatter(grid, [base + 1], zvec)
                plsc.store_scatter(grid, [base + 2], zvec)
                plsc.store_scatter(grid, [base + 4], zvec)
                return _c
            lax.fori_loop(0, ngrp, clean_body, 0)
        return _carry

    lax.fori_loop(0, B, batch_body, 0)

    gg = gg_sm[0]

    @pl.when(jnp.logical_and(gg >= 1, ((gg - 1) & 1) == 0))
    def _():
        pltpu.make_async_copy(stag.at[pl.ds(0, L)], out_hbm.at[idx0],
                              sem0).wait()

    @pl.when(jnp.logical_and(gg >= 1, ((gg - 1) & 1) == 1))
    def _():
        pltpu.make_async_copy(stag.at[pl.ds(L, L)], out_hbm.at[idx1],
                              sem1).wait()

    @pl.when(jnp.logical_and(gg >= 2, (gg & 1) == 0))
    def _():
        pltpu.make_async_copy(stag.at[pl.ds(0, L)], out_hbm.at[idx0],
                              sem0).wait()

    @pl.when(jnp.logical_and(gg >= 2, (gg & 1) == 1))
    def _():
        pltpu.make_async_copy(stag.at[pl.ds(L, L)], out_hbm.at[idx1],
                              sem1).wait()


def kernel(e, pos, species, feed_map, alive):
    xy = pos.reshape(B * 2 * N)
    sp = species.reshape(B * 2 * N)
    e2 = e.reshape(B * N)
    al = alive.reshape(B * N)
    out = _sc_world_model(xy, sp, e2, al, feed_map)
    return out[:, :PATCH].reshape(B, N, C, P, P)


# speedup vs baseline: 1.1182x; 1.1182x over previous
import functools

import jax
import jax.numpy as jnp
from jax import lax
from jax.experimental import pallas as pl
from jax.experimental.pallas import tpu as pltpu
from jax.experimental.pallas import tpu_sc as plsc

B, N, H, W, R, P, C = 8, 8192, 512, 512, 2, 5, 5
NC, NS, L = 2, 16, 16
NT = NC * NS
ROWS = H // NT
GR = ROWS + 2 * R
WP = W + 2 * R
GSZ = GR * WP * C
PATCH = C * P * P
OUTW = 128
NGRP = N // L

_mesh = plsc.VectorSubcoreMesh(
    core_axis_name="c", subcore_axis_name="s", num_cores=NC, num_subcores=NS)


@functools.partial(
    pl.kernel,
    out_type=jax.ShapeDtypeStruct((B * N, OUTW), jnp.float32),
    mesh=_mesh,
    compiler_params=pltpu.CompilerParams(
        needs_layout_passes=False, use_tc_tiling_on_sc=False),
    scratch_types=[
        pltpu.VMEM((GSZ,), jnp.float32),
        pltpu.VMEM((2 * N,), jnp.int32),
        pltpu.VMEM((2 * N,), jnp.float32),
        pltpu.VMEM((N,), jnp.float32),
        pltpu.VMEM((N,), jnp.float32),
        pltpu.VMEM((N,), jnp.int32),
        pltpu.VMEM((GR * W,), jnp.float32),
        pltpu.VMEM((2 * L, OUTW), jnp.float32),
        pltpu.VMEM((L, OUTW + 1), jnp.float32),
        pltpu.VMEM((L,), jnp.int32),
        pltpu.VMEM((L,), jnp.int32),
        pltpu.SMEM((1,), jnp.int32),
        pltpu.SMEM((1,), jnp.int32),
        pltpu.SemaphoreType.DMA,
        pltpu.SemaphoreType.DMA,
        pltpu.SemaphoreType.DMA,
    ],
)
def _sc_world_model(xy_hbm, sp_hbm, e_hbm, al_hbm, feed_hbm,
                    out_hbm, grid, xyv, spv, ev, alv, listv, feedv,
                    stag, stbf, idx0, idx1, cnt_sm, gg_sm, sem_in, sem0,
                    sem1):
    wid = lax.axis_index("s") * NC + lax.axis_index("c")
    lo = wid * ROWS
    iota = lax.iota(jnp.int32, L)
    iota2 = iota * 2
    ones = jnp.ones((L,), jnp.float32)
    zvec = jnp.zeros((L,), jnp.float32)
    gg_sm[0] = 0

    def out_wait(p_is_0_first):
        gg = gg_sm[0]
        p = gg & 1

        @pl.when(jnp.logical_and(gg >= 2, p == 0))
        def _():
            pltpu.make_async_copy(
                stag.at[pl.ds(0, L)], out_hbm.at[idx0], sem0).wait()

        @pl.when(jnp.logical_and(gg >= 2, p == 1))
        def _():
            pltpu.make_async_copy(
                stag.at[pl.ds(L, L)], out_hbm.at[idx1], sem1).wait()

    def batch_body(b, _carry):
        fires = [
            pltpu.async_copy(xy_hbm.at[b], xyv, sem_in),
            pltpu.async_copy(sp_hbm.at[b], spv, sem_in),
            pltpu.async_copy(e_hbm.at[b], ev, sem_in),
            pltpu.async_copy(al_hbm.at[b], alv, sem_in),
        ]
        feed_valid = []
        for l in range(GR):
            gr = lo - R + l
            valid = jnp.logical_and(gr >= 0, gr < H)
            feed_valid.append(valid)

            @pl.when(valid)
            def _(l=l, gr=gr):
                pltpu.async_copy(feed_hbm.at[b, gr],
                                 feedv.at[pl.ds(l * W, W)], sem_in)

        @pl.when(b == 0)
        def _():
            UZ = 25

            def zero_body(i, c):
                base = i * (UZ * L)
                for u in range(UZ):
                    grid[pl.ds(base + u * L, L)] = zvec
                return c
            lax.fori_loop(0, GSZ // (UZ * L), zero_body, 0)

        @pl.when(b > 0)
        def _():
            HZ = 5168
            UZ2 = 17

            def zero_body(i, c):
                base = i * (UZ2 * L)
                for u in range(UZ2):
                    grid[pl.ds(base + u * L, L)] = zvec
                    grid[pl.ds(GSZ - HZ + base + u * L, L)] = zvec
                return c
            lax.fori_loop(0, HZ // (UZ2 * L), zero_body, 0)

        for f in fires:
            f.wait()
        for l in range(GR):
            @pl.when(feed_valid[l])
            def _(l=l):
                pltpu.make_async_copy(feed_hbm.at[b, 0],
                                      feedv.at[pl.ds(l * W, W)], sem_in).wait()

        FCH = 8
        for l in range(GR):
            @pl.when(feed_valid[l])
            def _(l=l):
                for kk0 in range(0, W // L, FCH):
                    vs = [feedv[pl.ds(l * W + (kk0 + u) * L, L)]
                          for u in range(FCH)]
                    for u in range(FCH):
                        gidx = ((l * WP + R + (kk0 + u) * L) * C + 3
                                + iota * C)
                        plsc.store_scatter(grid, [gidx], vs[u])

        cnt_sm[0] = 0

        US = 4

        def scan_body(gq, _c):
            for u in range(US):
                off = (gq * US + u) * L
                idx2 = 2 * off + iota2
                xg = plsc.load_gather(xyv, [idx2])
                pre = jnp.logical_and(xg >= lo - R, xg < lo + ROWS + R)

                @pl.when(jnp.any(pre))
                def _(off=off, idx2=idx2, xg=xg, pre=pre):
                    yg = plsc.load_gather(xyv, [idx2 + 1])
                    ag = alv[pl.ds(off, L)]
                    m = jnp.logical_and(pre, ag > 0.0)
                    base = ((xg - (lo - R)) * WP + yg + R) * C
                    plsc.store_scatter(grid, [base],
                                       plsc.load_gather(spv, [idx2]), mask=m)
                    plsc.store_scatter(grid, [base + 1],
                                       plsc.load_gather(spv, [idx2 + 1]),
                                       mask=m)
                    plsc.store_scatter(grid, [base + 2], ev[pl.ds(off, L)],
                                       mask=m)
                    plsc.store_scatter(grid, [base + 4], ones, mask=m)
                    own = jnp.logical_and(xg >= lo, xg < lo + ROWS)
                    cnt = cnt_sm[0]
                    plsc.store_compressed(listv.at[pl.ds(cnt, L)],
                                          off + iota, mask=own)
                    pc = plsc.all_reduce_population_count(own)
                    cnt_sm[0] = cnt + pc[0]
            return _c

        lax.fori_loop(0, NGRP // US, scan_body, 0)

        kcnt = cnt_sm[0]
        ngrp = (kcnt + (L - 1)) >> 4

        def group_body(g, _c):
            out_wait(None)
            rem = kcnt - g * L
            safe_lane = jnp.minimum(iota, rem - 1)
            ids = plsc.load_gather(listv, [g * L + safe_lane])
            xg = plsc.load_gather(xyv, [ids * 2])
            yg = plsc.load_gather(xyv, [ids * 2 + 1])
            ag = plsc.load_gather(alv, [ids])
            p = gg_sm[0] & 1
            base = ((xg - lo) * WP + yg) * C
            GCH = 8
            for k0 in range(0, PATCH, GCH):
                ks = range(k0, min(k0 + GCH, PATCH))
                vs = []
                for k in ks:
                    c, ij = k // (P * P), k % (P * P)
                    i, j = ij // P, ij % P
                    d = (i * WP + j) * C + c
                    vs.append(plsc.load_gather(grid, [base + d]))
                for k, v in zip(ks, vs):
                    plsc.store_scatter(
                        stbf, [iota, jnp.full((L,), k, jnp.int32)], v * ag)
            for l0 in range(L):
                for u0 in range(OUTW // L):
                    stag[p * L + l0, pl.ds(u0 * L, L)] = (
                        stbf[l0, pl.ds(u0 * L, L)])
            rowids = b * N + ids

            @pl.when(p == 0)
            def _():
                idx0[...] = rowids
                pltpu.async_copy(stag.at[pl.ds(0, L)], out_hbm.at[idx0], sem0)

            @pl.when(p == 1)
            def _():
                idx1[...] = rowids
                pltpu.async_copy(stag.at[pl.ds(L, L)], out_hbm.at[idx1], sem1)

            gg_sm[0] = gg_sm[0] + 1
            return _c

        lax.fori_loop(0, ngrp, group_body, 0)

        @pl.when(b < B - 1)
        def _():
            def clean_body(g, _c):
                safe_lane = jnp.minimum(iota, kcnt - g * L - 1)
                ids = plsc.load_gather(listv, [g * L + safe_lane])
                xg = plsc.load_gather(xyv, [ids * 2])
                yg = plsc.load_gather(xyv, [ids * 2 + 1])
                base = ((xg - (lo - R)) * WP + yg + R) * C
                plsc.store_scatter(grid, [base], zvec)
                plsc.store_scatter(grid, [base + 1], zvec)
                plsc.store_scatter(grid, [base + 2], zvec)
                plsc.store_scatter(grid, [base + 4], zvec)
                return _c
            lax.fori_loop(0, ngrp, clean_body, 0)
        return _carry

    lax.fori_loop(0, B, batch_body, 0)

    gg = gg_sm[0]

    @pl.when(jnp.logical_and(gg >= 1, ((gg - 1) & 1) == 0))
    def _():
        pltpu.make_async_copy(stag.at[pl.ds(0, L)], out_hbm.at[idx0],
                              sem0).wait()

    @pl.when(jnp.logical_and(gg >= 1, ((gg - 1) & 1) == 1))
    def _():
        pltpu.make_async_copy(stag.at[pl.ds(L, L)], out_hbm.at[idx1],
                              sem1).wait()

    @pl.when(jnp.logical_and(gg >= 2, (gg & 1) == 0))
    def _():
        pltpu.make_async_copy(stag.at[pl.ds(0, L)], out_hbm.at[idx0],
                              sem0).wait()

    @pl.when(jnp.logical_and(gg >= 2, (gg & 1) == 1))
    def _():
        pltpu.make_async_copy(stag.at[pl.ds(L, L)], out_hbm.at[idx1],
                              sem1).wait()


def kernel(e, pos, species, feed_map, alive):
    xy = pos.reshape(B, 2 * N).astype(jnp.int32)
    sp = species.reshape(B, 2 * N)
    e2 = e.reshape(B, N)
    al = alive.reshape(B, N)
    out = _sc_world_model(xy, sp, e2, al, feed_map)
    return out[:, :PATCH].reshape(B, N, C, P, P)


# speedup vs baseline: 1.5714x; 1.4053x over previous
import functools

import jax
import jax.numpy as jnp
from jax import lax
from jax.experimental import pallas as pl
from jax.experimental.pallas import tpu as pltpu
from jax.experimental.pallas import tpu_sc as plsc

B, N, H, W, R, P, C = 8, 8192, 512, 512, 2, 5, 5
NC, NS, L = 2, 16, 16
NT = NC * NS
ROWS = H // NT
GR = ROWS + 2 * R
WP = W + 2 * R
GSZ = GR * WP * C
PATCH = C * P * P
OUTW = 128
NGRP = N // L

_mesh = plsc.VectorSubcoreMesh(
    core_axis_name="c", subcore_axis_name="s", num_cores=NC, num_subcores=NS)


@functools.partial(
    pl.kernel,
    out_type=(jax.ShapeDtypeStruct((B * N, OUTW), jnp.float32),
              jax.ShapeDtypeStruct((NC * 16 * 16 * 512,), jnp.float32),
              jax.ShapeDtypeStruct((NC * 16 * 16,), jnp.int32)),
    mesh=_mesh,
    compiler_params=pltpu.CompilerParams(
        needs_layout_passes=False, use_tc_tiling_on_sc=False),
    scratch_types=[
        pltpu.VMEM((GSZ,), jnp.float32),
        pltpu.VMEM((2 * N,), jnp.int32),
        pltpu.VMEM((2 * N,), jnp.float32),
        pltpu.VMEM((N,), jnp.float32),
        pltpu.VMEM((N,), jnp.float32),
        pltpu.VMEM((N,), jnp.int32),
        pltpu.VMEM((GR * W,), jnp.float32),
        pltpu.VMEM((2 * L, OUTW), jnp.float32),
        pltpu.VMEM((L, OUTW + 1), jnp.float32),
        pltpu.VMEM((L,), jnp.int32),
        pltpu.VMEM((L,), jnp.int32),
        pltpu.VMEM((16,), jnp.int32),
        pltpu.VMEM((256,), jnp.int32),
        pltpu.SMEM((1,), jnp.int32),
        pltpu.SMEM((1,), jnp.int32),
        pltpu.SemaphoreType.DMA,
        pltpu.SemaphoreType.DMA,
        pltpu.SemaphoreType.DMA,
    ],
)
def _sc_world_model(xy_hbm, sp_hbm, e_hbm, al_hbm, feed_hbm,
                    out_hbm, mbox_sh, counts_sh, grid, xyv, spv, ev, alv,
                    listv, feedv, stag, stbf, idx0, idx1, ctrv, countsv,
                    cnt_sm, gg_sm, sem_in, sem0, sem1):
    sidx = lax.axis_index("s")
    cidx = lax.axis_index("c")
    wid = sidx * NC + cidx
    lo = wid * ROWS
    iota = lax.iota(jnp.int32, L)
    iota2 = iota * 2
    ones = jnp.ones((L,), jnp.float32)
    zvec = jnp.zeros((L,), jnp.float32)
    gg_sm[0] = 0

    def out_wait(p_is_0_first):
        gg = gg_sm[0]
        p = gg & 1

        @pl.when(jnp.logical_and(gg >= 2, p == 0))
        def _():
            pltpu.make_async_copy(
                stag.at[pl.ds(0, L)], out_hbm.at[idx0], sem0).wait()

        @pl.when(jnp.logical_and(gg >= 2, p == 1))
        def _():
            pltpu.make_async_copy(
                stag.at[pl.ds(L, L)], out_hbm.at[idx1], sem1).wait()

    def batch_body(b, _carry):
        fires = [
            pltpu.async_copy(xy_hbm.at[b], xyv, sem_in),
            pltpu.async_copy(sp_hbm.at[b], spv, sem_in),
            pltpu.async_copy(e_hbm.at[b], ev, sem_in),
            pltpu.async_copy(al_hbm.at[b], alv, sem_in),
        ]
        feed_valid = []
        for l in range(GR):
            gr = lo - R + l
            valid = jnp.logical_and(gr >= 0, gr < H)
            feed_valid.append(valid)

            @pl.when(valid)
            def _(l=l, gr=gr):
                pltpu.async_copy(feed_hbm.at[b, gr],
                                 feedv.at[pl.ds(l * W, W)], sem_in)

        @pl.when(b == 0)
        def _():
            UZ = 25

            def zero_body(i, c):
                base = i * (UZ * L)
                for u in range(UZ):
                    grid[pl.ds(base + u * L, L)] = zvec
                return c
            lax.fori_loop(0, GSZ // (UZ * L), zero_body, 0)

        @pl.when(b > 0)
        def _():
            HZ = 5168
            UZ2 = 17

            def zero_body(i, c):
                base = i * (UZ2 * L)
                for u in range(UZ2):
                    grid[pl.ds(base + u * L, L)] = zvec
                    grid[pl.ds(GSZ - HZ + base + u * L, L)] = zvec
                return c
            lax.fori_loop(0, HZ // (UZ2 * L), zero_body, 0)

        for f in fires:
            f.wait()
        for l in range(GR):
            @pl.when(feed_valid[l])
            def _(l=l):
                pltpu.make_async_copy(feed_hbm.at[b, 0],
                                      feedv.at[pl.ds(l * W, W)], sem_in).wait()

        FCH = 8
        for l in range(GR):
            @pl.when(feed_valid[l])
            def _(l=l):
                for kk0 in range(0, W // L, FCH):
                    vs = [feedv[pl.ds(l * W + (kk0 + u) * L, L)]
                          for u in range(FCH)]
                    for u in range(FCH):
                        gidx = ((l * WP + R + (kk0 + u) * L) * C + 3
                                + iota * C)
                        plsc.store_scatter(grid, [gidx], vs[u])

        ctrv[...] = jnp.zeros((L,), jnp.int32)

        def route_body(g, _c):
            off = sidx * (N // NS) + g * L
            idx2 = 2 * off + iota2
            xg = plsc.load_gather(xyv, [idx2])
            r0 = xg >> 4
            xm = xg & 15
            is_own = (r0 & 1) == cidx
            d_h = jnp.where(xm < R, r0 - 1, r0 + 1)
            halo_ok = jnp.logical_or(xm < R, xm >= ROWS - R)
            d = jnp.where(is_own, r0, d_h)
            valid = jnp.logical_or(
                is_own,
                jnp.logical_and(halo_ok,
                                jnp.logical_and(d_h >= 0, d_h < NT)))
            ds = jnp.where(valid, d >> 1, 0)
            rank, lastm = plsc.scan_count(ds, mask=valid)
            basec = plsc.load_gather(ctrv, [ds])
            pos = basec + rank - 1
            plsc.store_scatter(feedv, [ds * 512 + pos],
                               (off + iota).astype(jnp.float32), mask=valid)
            plsc.store_scatter(ctrv, [ds], pos + 1, mask=lastm)
            return _c

        lax.fori_loop(0, (N // NS) // L, route_body, 0)

        def tsend_body(dd, _c):
            pltpu.async_copy(
                feedv.at[pl.ds(dd * 512, 512)],
                mbox_sh.at[pl.ds(((cidx * 16 + dd) * 16 + sidx) * 512, 512)],
                sem_in)
            return _c

        lax.fori_loop(0, NS, tsend_body, 0)
        pltpu.async_copy(
            ctrv, counts_sh.at[pl.ds((cidx * 16 + sidx) * 16, 16)], sem_in)

        def tsendw_body(dd, _c):
            pltpu.make_async_copy(
                feedv.at[pl.ds(dd * 512, 512)],
                mbox_sh.at[pl.ds(((cidx * 16 + dd) * 16 + sidx) * 512, 512)],
                sem_in).wait()
            return _c

        lax.fori_loop(0, NS, tsendw_body, 0)
        pltpu.make_async_copy(
            ctrv, counts_sh.at[pl.ds((cidx * 16 + sidx) * 16, 16)],
            sem_in).wait()
        plsc.subcore_barrier()

        def trecv_body(ss, _c):
            pltpu.async_copy(
                mbox_sh.at[pl.ds(((cidx * 16 + sidx) * 16 + ss) * 512, 512)],
                feedv.at[pl.ds(ss * 512, 512)], sem_in)
            return _c

        lax.fori_loop(0, NS, trecv_body, 0)
        pltpu.async_copy(counts_sh.at[pl.ds(cidx * 256, 256)], countsv,
                         sem_in)

        def trecvw_body(ss, _c):
            pltpu.make_async_copy(
                mbox_sh.at[pl.ds(((cidx * 16 + sidx) * 16 + ss) * 512, 512)],
                feedv.at[pl.ds(ss * 512, 512)], sem_in).wait()
            return _c

        lax.fori_loop(0, NS, trecvw_body, 0)
        pltpu.make_async_copy(counts_sh.at[pl.ds(cidx * 256, 256)], countsv,
                              sem_in).wait()
        plsc.subcore_barrier()

        cnt_sm[0] = 0
        ccol = plsc.load_gather(countsv, [iota * 16 + sidx])

        def src_body(ss, _c):
            cnt_sd = jnp.max(jnp.where(iota == ss, ccol, 0))

            def dst_body(g, _cc):
                rem = cnt_sd - g * L
                ids = plsc.load_gather(
                    feedv,
                    [ss * 512 + g * L + jnp.minimum(iota, rem - 1)]
                ).astype(jnp.int32)
                idx2b = ids * 2
                xg = plsc.load_gather(xyv, [idx2b])
                yg = plsc.load_gather(xyv, [idx2b + 1])
                ag = plsc.load_gather(alv, [ids])
                s0g = plsc.load_gather(spv, [idx2b])
                s1g = plsc.load_gather(spv, [idx2b + 1])
                eg = plsc.load_gather(ev, [ids])
                m = ag > 0.0
                base = ((xg - (lo - R)) * WP + yg + R) * C
                plsc.store_scatter(grid, [base], s0g, mask=m)
                plsc.store_scatter(grid, [base + 1], s1g, mask=m)
                plsc.store_scatter(grid, [base + 2], eg, mask=m)
                plsc.store_scatter(grid, [base + 4], ones, mask=m)
                own = jnp.logical_and(
                    jnp.logical_and(xg >= lo, xg < lo + ROWS),
                    iota < rem)
                cnt = cnt_sm[0]
                plsc.store_compressed(listv.at[pl.ds(cnt, L)], ids, mask=own)
                pc = plsc.all_reduce_population_count(own)
                cnt_sm[0] = cnt + pc[0]
                return _cc

            lax.fori_loop(0, (cnt_sd + (L - 1)) >> 4, dst_body, 0)
            return _c

        lax.fori_loop(0, NS, src_body, 0)

        kcnt = cnt_sm[0]
        ngrp = (kcnt + (L - 1)) >> 4

        def group_body(g, _c):
            out_wait(None)
            rem = kcnt - g * L
            safe_lane = jnp.minimum(iota, rem - 1)
            ids = plsc.load_gather(listv, [g * L + safe_lane])
            xg = plsc.load_gather(xyv, [ids * 2])
            yg = plsc.load_gather(xyv, [ids * 2 + 1])
            ag = plsc.load_gather(alv, [ids])
            p = gg_sm[0] & 1
            base = ((xg - lo) * WP + yg) * C
            GCH = 8
            for k0 in range(0, PATCH, GCH):
                ks = range(k0, min(k0 + GCH, PATCH))
                vs = []
                for k in ks:
                    c, ij = k // (P * P), k % (P * P)
                    i, j = ij // P, ij % P
                    d = (i * WP + j) * C + c
                    vs.append(plsc.load_gather(grid, [base + d]))
                for k, v in zip(ks, vs):
                    plsc.store_scatter(
                        stbf, [iota, jnp.full((L,), k, jnp.int32)], v * ag)
            for l0 in range(L):
                for u0 in range(OUTW // L):
                    stag[p * L + l0, pl.ds(u0 * L, L)] = (
                        stbf[l0, pl.ds(u0 * L, L)])
            rowids = b * N + ids

            @pl.when(p == 0)
            def _():
                idx0[...] = rowids
                pltpu.async_copy(stag.at[pl.ds(0, L)], out_hbm.at[idx0], sem0)

            @pl.when(p == 1)
            def _():
                idx1[...] = rowids
                pltpu.async_copy(stag.at[pl.ds(L, L)], out_hbm.at[idx1], sem1)

            gg_sm[0] = gg_sm[0] + 1
            return _c

        lax.fori_loop(0, ngrp, group_body, 0)

        @pl.when(b < B - 1)
        def _():
            def clean_body(g, _c):
                safe_lane = jnp.minimum(iota, kcnt - g * L - 1)
                ids = plsc.load_gather(listv, [g * L + safe_lane])
                xg = plsc.load_gather(xyv, [ids * 2])
                yg = plsc.load_gather(xyv, [ids * 2 + 1])
                base = ((xg - (lo - R)) * WP + yg + R) * C
                plsc.store_scatter(grid, [base], zvec)
                plsc.store_scatter(grid, [base + 1], zvec)
                plsc.store_scatter(grid, [base + 2], zvec)
                plsc.store_scatter(grid, [base + 4], zvec)
                return _c
            lax.fori_loop(0, ngrp, clean_body, 0)
        return _carry

    lax.fori_loop(0, B, batch_body, 0)

    gg = gg_sm[0]

    @pl.when(jnp.logical_and(gg >= 1, ((gg - 1) & 1) == 0))
    def _():
        pltpu.make_async_copy(stag.at[pl.ds(0, L)], out_hbm.at[idx0],
                              sem0).wait()

    @pl.when(jnp.logical_and(gg >= 1, ((gg - 1) & 1) == 1))
    def _():
        pltpu.make_async_copy(stag.at[pl.ds(L, L)], out_hbm.at[idx1],
                              sem1).wait()

    @pl.when(jnp.logical_and(gg >= 2, (gg & 1) == 0))
    def _():
        pltpu.make_async_copy(stag.at[pl.ds(0, L)], out_hbm.at[idx0],
                              sem0).wait()

    @pl.when(jnp.logical_and(gg >= 2, (gg & 1) == 1))
    def _():
        pltpu.make_async_copy(stag.at[pl.ds(L, L)], out_hbm.at[idx1],
                              sem1).wait()


def kernel(e, pos, species, feed_map, alive):
    xy = pos.reshape(B, 2 * N).astype(jnp.int32)
    sp = species.reshape(B, 2 * N)
    e2 = e.reshape(B, N)
    al = alive.reshape(B, N)
    out, _mbox, _cnts = _sc_world_model(xy, sp, e2, al, feed_map)
    return out[:, :PATCH].reshape(B, N, C, P, P)
